# baseline (speedup 1.0000x reference)
import functools

import jax
import jax.numpy as jnp
from jax import lax
from jax.experimental import pallas as pl
from jax.experimental.pallas import tpu as pltpu
from jax.experimental.pallas import tpu_sc as plsc

_ALPHA = 0.95
_NC = 2
_NS = 16
_NW = _NC * _NS
_L = 16


def kernel(teacher_logits, true_labels):
    b, c = teacher_logits.shape
    rows_w = b // _NW
    nr = 8
    nfull = c // _L
    rem = c - nfull * _L
    labels = true_labels.astype(jnp.int32)
    mesh = plsc.VectorSubcoreMesh(core_axis_name="c", subcore_axis_name="s")

    @functools.partial(
        pl.kernel,
        out_type=jax.ShapeDtypeStruct((b, c), jnp.float32),
        mesh=mesh,
        compiler_params=pltpu.CompilerParams(needs_layout_passes=False),
        scratch_types=[
            pltpu.VMEM((rows_w + _L,), jnp.int32),
            pltpu.VMEM((nr + 1, c), jnp.float32),
            pltpu.VMEM((nr + 1, c), jnp.float32),
        ],
    )
    def loca_sc(x_hbm, lab_hbm, out_hbm, lab_v, x_v, o_v):
        wid = lax.axis_index("s") * _NC + lax.axis_index("c")
        base = wid * rows_w
        pltpu.sync_copy(lab_hbm.at[pl.ds(base, rows_w)],
                        lab_v.at[pl.ds(0, rows_w)])

        lane = lax.iota(jnp.int32, _L)

        def row_body(r, chunk):
            acc = jnp.zeros((_L,), jnp.float32)
            for j in range(nfull):
                acc = acc + x_v[r, pl.ds(j * _L, _L)]
            if rem:
                tail = x_v[r, pl.ds(c - _L, _L)]
                acc = acc + jnp.where(lane >= _L - rem, tail, 0.0)
            rs = jnp.sum(acc)
            y = lab_v[pl.ds(chunk * nr + r, _L)][0]
            cidx = (y // _L) * _L
            off = y - cidx
            tvec = x_v[r, pl.ds(cidx, _L)]
            t = jnp.sum(jnp.where(lane == off, tvec, 0.0))
            t16 = jnp.full((_L,), t)
            rs16 = jnp.full((_L,), rs)
            s16 = _ALPHA / ((1.0 - 2.0 * t16) + rs16)
            tv16 = (1.0 - s16 * rs16) + s16 * t16
            for j in range(nfull):
                o_v[r, pl.ds(j * _L, _L)] = s16 * x_v[r, pl.ds(j * _L, _L)]
            if rem:
                o_v[r, pl.ds(c - _L, _L)] = s16 * x_v[r, pl.ds(c - _L, _L)]
            cur = o_v[r, pl.ds(cidx, _L)]
            o_v[r, pl.ds(cidx, _L)] = jnp.where(lane == off, tv16, cur)
            return chunk

        def chunk_body(ci, carry):
            row0 = base + ci * nr
            pltpu.sync_copy(x_hbm.at[pl.ds(row0, nr)],
                            x_v.at[pl.ds(0, nr)])
            lax.fori_loop(0, nr, row_body, ci)
            pltpu.sync_copy(o_v.at[pl.ds(0, nr)],
                            out_hbm.at[pl.ds(row0, nr)])
            return carry

        lax.fori_loop(0, rows_w // nr, chunk_body, 0)

    return loca_sc(teacher_logits, labels)


# speedup vs baseline: 2.4182x; 2.4182x over previous
import jax
import jax.numpy as jnp
from jax import lax
from jax.experimental import pallas as pl

_ALPHA = 0.95


def _row_sum_exact_order(x):
    r, c = x.shape
    xt = x.T
    acc = xt[0:8, :]
    for k in range(1, c // 8):
        acc = acc + xt[8 * k : 8 * k + 8, :]
    a = [acc[i : i + 1, :] for i in range(8)]
    rs_t = ((a[0] + a[4]) + (a[2] + a[6])) + ((a[1] + a[5]) + (a[3] + a[7]))
    return rs_t.T


def _loca_body(x_ref, lab_ref, out_ref):
    x = x_ref[...]
    lab = lab_ref[...]
    r, c = x.shape
    col = lax.broadcasted_iota(jnp.int32, (r, c), 1)
    mask = col == lab
    rs = _row_sum_exact_order(x)
    t = jnp.sum(jnp.where(mask, x, 0.0), axis=1, keepdims=True)
    s = _ALPHA / (1.0 - 2.0 * t + rs)
    tv = 1.0 - s * rs + s * t
    out_ref[...] = jnp.where(mask, tv, s * x)


def kernel(teacher_logits, true_labels):
    b, c = teacher_logits.shape
    rows = 2048
    lab2 = true_labels.astype(jnp.int32).reshape(b, 1)
    return pl.pallas_call(
        _loca_body,
        grid=(b // rows,),
        in_specs=[
            pl.BlockSpec((rows, c), lambda i: (i, 0)),
            pl.BlockSpec((rows, 1), lambda i: (i, 0)),
        ],
        out_specs=pl.BlockSpec((rows, c), lambda i: (i, 0)),
        out_shape=jax.ShapeDtypeStruct((b, c), jnp.float32),
    )(teacher_logits, lab2)
